# baseline (speedup 1.0000x reference)
import functools

import jax
import jax.numpy as jnp
from jax import lax
from jax.experimental import pallas as pl
from jax.experimental.pallas import tpu as pltpu
from jax.experimental.pallas import tpu_sc as plsc

B = 4
N = 8192
E = 131072
D = 128
DC = 64
NC, NS = 2, 16
BLK = 2048

C_AGG = 128
EP = E // NS
NCH_E = EP // C_AGG
RT = N // NS
NCH_EMB = 16

_MESH = plsc.VectorSubcoreMesh(
    core_axis_name="c", subcore_axis_name="s", num_cores=NC, num_subcores=NS)
_SC_PARAMS = pltpu.CompilerParams(use_tc_tiling_on_sc=False)

f32 = jnp.float32
i32 = jnp.int32


def _zero_fill(zbuf, rows, width):
    zero16 = jnp.zeros((16,), f32)

    def zrow(r, carry):
        for cc in range(width // 16):
            zbuf[r, pl.ds(cc * 16, 16)] = zero16
        return carry

    lax.fori_loop(0, rows, zrow, 0)


def _zero_accum(agg_s, zbuf, s, zrows):
    for i in range(RT // zrows):
        pltpu.sync_copy(zbuf, agg_s.at[pl.ds(s * RT + i * zrows, zrows)])


def _agg_pass(h_hbm, agg_s, soff, didx, gbufa, gbufb, sga, sgb, ssa, ssb,
              deg_tap=None):
    del ssa, ssb
    pltpu.async_copy(h_hbm.at[soff.at[0]], gbufa, sga)

    def epair(p, carry):
        i0 = 2 * p
        pltpu.async_copy(h_hbm.at[soff.at[i0 + 1]], gbufb, sgb)
        pltpu.make_async_copy(h_hbm.at[soff.at[i0]], gbufa, sga).wait()
        pltpu.sync_copy(gbufa, agg_s.at[didx.at[i0]], add=True)
        if deg_tap is not None:
            deg_tap(i0)

        @pl.when(i0 + 2 < NCH_E)
        def _():
            pltpu.async_copy(h_hbm.at[soff.at[i0 + 2]], gbufa, sga)

        pltpu.make_async_copy(h_hbm.at[soff.at[i0 + 1]], gbufb, sgb).wait()
        pltpu.sync_copy(gbufb, agg_s.at[didx.at[i0 + 1]], add=True)
        if deg_tap is not None:
            deg_tap(i0 + 1)
        return carry

    lax.fori_loop(0, NCH_E // 2, epair, 0)


def _addoff(soff, delta):
    def body(t, carry):
        i = t // (C_AGG // 16)
        j = t - i * (C_AGG // 16)
        sl = pl.ds(j * 16, 16)
        soff[i, sl] = soff[i, sl] + delta
        return carry

    lax.fori_loop(0, NCH_E * (C_AGG // 16), body, 0)


@functools.partial(
    pl.kernel,
    out_type=(jax.ShapeDtypeStruct((B * N, DC), f32),
              jax.ShapeDtypeStruct((N, 16), f32),
              jax.ShapeDtypeStruct((B * N, DC), f32)),
    mesh=_MESH,
    scratch_types=[
        pltpu.VMEM((64, C_AGG), i32),
        pltpu.VMEM((64, C_AGG), i32),
        pltpu.VMEM((C_AGG, DC), f32),
        pltpu.VMEM((C_AGG, DC), f32),
        pltpu.VMEM((C_AGG, 16), f32),
        pltpu.VMEM((C_AGG, 16), f32),
        pltpu.VMEM((32, DC), f32),
        pltpu.VMEM((16, DC), f32),
        pltpu.VMEM((C_AGG, 16), f32),
        pltpu.VMEM((64, 16), f32),
        pltpu.VMEM_SHARED((N, DC), f32),
        pltpu.VMEM_SHARED((N, 16), f32),
        pltpu.SemaphoreType.DMA,
        pltpu.SemaphoreType.DMA,
        pltpu.SemaphoreType.DMA,
        pltpu.SemaphoreType.DMA,
    ],
    compiler_params=_SC_PARAMS,
)
def _embed_agg0(emb_hbm, xl_hbm, ei_hbm, xcat_hbm, deg_hbm, agg_hbm,
                soff, didx, gbufa, gbufb, ebufa, ebufb, ebuf2, zbuf,
                ones_v, zdeg, agg_s, deg_s, sema, semb, semc, semd):
    c = lax.axis_index("c")
    s = lax.axis_index("s")

    _zero_fill(zbuf, 16, DC)
    _zero_accum(agg_s, zbuf, s, 16)
    pltpu.sync_copy(ei_hbm.at[1, s], didx)

    @pl.when(c == 0)
    def _():
        ones16 = jnp.ones((16,), f32)
        zero16 = jnp.zeros((16,), f32)

        def fill(r, carry):
            ones_v[r, pl.ds(0, 16)] = ones16
            return carry

        lax.fori_loop(0, C_AGG, fill, 0)

        def zfill(r, carry):
            zdeg[r, pl.ds(0, 16)] = zero16
            return carry

        lax.fori_loop(0, 64, zfill, 0)
        for i in range(RT // 64):
            pltpu.sync_copy(zdeg, deg_s.at[pl.ds(s * RT + i * 64, 64)])

    plsc.subcore_barrier()

    for local_b in range(2):
        b = 2 * c + local_b
        boff = b * N

        pltpu.sync_copy(xl_hbm.at[b, s], soff.at[pl.ds(0, 16)])

        def estep(i, ebuf, sem):
            pltpu.make_async_copy(
                emb_hbm.at[soff.at[i]], ebuf, sem).wait()

            def rl(t, carry):
                r = t // 4
                cc = t - r * 4
                ebuf2[r, pl.ds(cc * 16, 16)] = ebuf[t, pl.ds(0, 16)]
                return carry

            lax.fori_loop(0, C_AGG, rl, 0)
            pltpu.sync_copy(
                ebuf2, xcat_hbm.at[pl.ds(boff + s * RT + i * 32, 32)])

        pltpu.async_copy(emb_hbm.at[soff.at[0]], ebufa, sema)

        def embpair(p, carry):
            i0 = 2 * p
            pltpu.async_copy(emb_hbm.at[soff.at[i0 + 1]], ebufb, semb)
            estep(i0, ebufa, sema)

            @pl.when(i0 + 2 < NCH_EMB)
            def _():
                pltpu.async_copy(emb_hbm.at[soff.at[i0 + 2]], ebufa, sema)

            estep(i0 + 1, ebufb, semb)
            return carry

        lax.fori_loop(0, NCH_EMB // 2, embpair, 0)
        plsc.subcore_barrier()

        pltpu.sync_copy(ei_hbm.at[0, s], soff)
        _addoff(soff, boff)

        if local_b == 0:
            def deg_tap(i):
                @pl.when(c == 0)
                def _():
                    pltpu.sync_copy(ones_v, deg_s.at[didx.at[i]], add=True)
        else:
            deg_tap = None
        _agg_pass(xcat_hbm, agg_s, soff, didx, gbufa, gbufb, sema, semb,
                  semc, semd, deg_tap)
        plsc.subcore_barrier()
        pltpu.sync_copy(agg_s.at[pl.ds(s * RT, RT)],
                        agg_hbm.at[pl.ds(boff + s * RT, RT)])
        if local_b == 0:
            @pl.when(c == 0)
            def _():
                pltpu.sync_copy(deg_s.at[pl.ds(s * RT, RT)],
                                deg_hbm.at[pl.ds(s * RT, RT)])
            _zero_accum(agg_s, zbuf, s, 16)
            plsc.subcore_barrier()


@functools.partial(
    pl.kernel,
    out_type=jax.ShapeDtypeStruct((B * N, D), f32),
    mesh=_MESH,
    scratch_types=[
        pltpu.VMEM((64, C_AGG), i32),
        pltpu.VMEM((64, C_AGG), i32),
        pltpu.VMEM((C_AGG, D), f32),
        pltpu.VMEM((C_AGG, D), f32),
        pltpu.VMEM((16, D), f32),
        pltpu.VMEM_SHARED((N, D), f32),
        pltpu.SemaphoreType.DMA,
        pltpu.SemaphoreType.DMA,
        pltpu.SemaphoreType.DMA,
        pltpu.SemaphoreType.DMA,
    ],
    compiler_params=_SC_PARAMS,
)
def _agg(h_hbm, ei_hbm, agg_hbm, soff, didx, gbufa, gbufb, zbuf,
         agg_s, sema, semb, semc, semd):
    c = lax.axis_index("c")
    s = lax.axis_index("s")

    _zero_fill(zbuf, 16, D)
    _zero_accum(agg_s, zbuf, s, 16)
    pltpu.sync_copy(ei_hbm.at[0, s], soff)
    pltpu.sync_copy(ei_hbm.at[1, s], didx)
    plsc.subcore_barrier()

    for local_b in range(2):
        b = 2 * c + local_b
        boff = b * N
        _addoff(soff, boff if local_b == 0 else N)
        _agg_pass(h_hbm, agg_s, soff, didx, gbufa, gbufb, sema, semb,
                  semc, semd)
        plsc.subcore_barrier()
        pltpu.sync_copy(agg_s.at[pl.ds(s * RT, RT)],
                        agg_hbm.at[pl.ds(boff + s * RT, RT)])
        if local_b == 0:
            _zero_accum(agg_s, zbuf, s, 16)
            plsc.subcore_barrier()


def _layer0_body(xc_ref, agg_ref, deg_ref, w64_ref, wrole_ref, role1_ref,
                 linb_ref, wl_ref, bl_ref, wr_ref, out_ref, inv_ref):
    degcol = deg_ref[...][:, 0:1]
    inv = 1.0 / jnp.maximum(degcol, 1.0)
    gate = jnp.where(degcol > 0.0, 1.0, 0.0)
    beff = linb_ref[...] + jnp.dot(
        role1_ref[...], wrole_ref[...], preferred_element_type=f32)
    mean_c = agg_ref[...] * inv
    x0 = jnp.dot(xc_ref[...], w64_ref[...], preferred_element_type=f32) + beff
    m0 = (jnp.dot(mean_c, w64_ref[...], preferred_element_type=f32)
          + gate * beff)
    h = (jnp.dot(m0, wl_ref[...], preferred_element_type=f32) + bl_ref[...]
         + jnp.dot(x0, wr_ref[...], preferred_element_type=f32))
    out_ref[...] = jnp.maximum(h, 0.0)
    inv_ref[...] = jnp.broadcast_to(inv, (BLK, D))


def _layer0(xc, agg, deg2, w64T, wroleT, role1, linb2, wlT, bl2, wrT):
    nb = (B * N) // BLK
    return pl.pallas_call(
        _layer0_body,
        grid=(nb,),
        in_specs=[
            pl.BlockSpec((BLK, DC), lambda j: (j, 0)),
            pl.BlockSpec((BLK, DC), lambda j: (j, 0)),
            pl.BlockSpec((BLK, 16), lambda j: (lax.rem(j, N // BLK), 0)),
            pl.BlockSpec((DC, D), lambda j: (0, 0)),
            pl.BlockSpec((16, D), lambda j: (0, 0)),
            pl.BlockSpec((1, 16), lambda j: (0, 0)),
            pl.BlockSpec((1, D), lambda j: (0, 0)),
            pl.BlockSpec((D, D), lambda j: (0, 0)),
            pl.BlockSpec((1, D), lambda j: (0, 0)),
            pl.BlockSpec((D, D), lambda j: (0, 0)),
        ],
        out_specs=[
            pl.BlockSpec((BLK, D), lambda j: (j, 0)),
            pl.BlockSpec((BLK, D), lambda j: (lax.rem(j, N // BLK), 0)),
        ],
        out_shape=[jax.ShapeDtypeStruct((B * N, D), f32),
                   jax.ShapeDtypeStruct((N, D), f32)],
    )(xc, agg, deg2, w64T, wroleT, role1, linb2, wlT, bl2, wrT)


def _layer1_head_body(x_ref, agg_ref, inv_ref, wl_ref, bl_ref, wr_ref,
                      w1_ref, b1_ref, w2_ref, b2_ref, w3_ref, b3_ref,
                      out_ref):
    mean = agg_ref[...] * inv_ref[...]
    h = (jnp.dot(mean, wl_ref[...], preferred_element_type=f32) + bl_ref[...]
         + jnp.dot(x_ref[...], wr_ref[...], preferred_element_type=f32))
    h = jnp.maximum(h, 0.0)
    h = jnp.maximum(
        jnp.dot(h, w1_ref[...], preferred_element_type=f32) + b1_ref[...], 0.0)
    h = jnp.maximum(
        jnp.dot(h, w2_ref[...], preferred_element_type=f32) + b2_ref[...], 0.0)
    out = (jnp.dot(h, w3_ref[...], preferred_element_type=f32)
           + b3_ref[...])
    out_ref[...] = out.reshape(1, BLK, 2)


def _layer1_head(x, agg, invb, wlT, bl2, wrT, w1T, b12, w2T, b22, w3T, b32):
    nbn = N // BLK
    return pl.pallas_call(
        _layer1_head_body,
        grid=(B, nbn),
        in_specs=[
            pl.BlockSpec((BLK, D), lambda b, j: (b * nbn + j, 0)),
            pl.BlockSpec((BLK, D), lambda b, j: (b * nbn + j, 0)),
            pl.BlockSpec((BLK, D), lambda b, j: (j, 0)),
            pl.BlockSpec((D, D), lambda b, j: (0, 0)),
            pl.BlockSpec((1, D), lambda b, j: (0, 0)),
            pl.BlockSpec((D, D), lambda b, j: (0, 0)),
            pl.BlockSpec((D, D), lambda b, j: (0, 0)),
            pl.BlockSpec((1, D), lambda b, j: (0, 0)),
            pl.BlockSpec((D, D), lambda b, j: (0, 0)),
            pl.BlockSpec((1, D), lambda b, j: (0, 0)),
            pl.BlockSpec((D, 2), lambda b, j: (0, 0)),
            pl.BlockSpec((1, 2), lambda b, j: (0, 0)),
        ],
        out_specs=pl.BlockSpec((1, BLK, 2), lambda b, j: (b, j, 0)),
        out_shape=jax.ShapeDtypeStruct((B, N, 2), f32),
    )(x, agg, invb, wlT, bl2, wrT, w1T, b12, w2T, b22, w3T, b32)


def kernel(x_layout, x_role, edge_index, role_emb, layout_emb, lin_W, lin_b,
           c0_Wl, c0_bl, c0_Wr, c1_Wl, c1_bl, c1_Wr,
           d1_W, d1_b, d2_W, d2_b, d3_W, d3_b):
    del x_role
    xl3 = x_layout.reshape(B, NS, NCH_EMB, C_AGG)
    ei4 = edge_index.reshape(2, NS, NCH_E, C_AGG)

    xcat, deg2, agg0 = _embed_agg0(layout_emb, xl3, ei4)
    h1, invb = _layer0(xcat, agg0, deg2, lin_W[:, :DC].T, lin_W[:, DC:80].T,
                       role_emb[1:2, :], lin_b.reshape(1, D),
                       c0_Wl.T, c0_bl.reshape(1, D), c0_Wr.T)
    agg1 = _agg(h1, ei4)
    return _layer1_head(h1, agg1, invb, c1_Wl.T, c1_bl.reshape(1, D),
                        c1_Wr.T, d1_W.T, d1_b.reshape(1, D), d2_W.T,
                        d2_b.reshape(1, D), d3_W.T, d3_b.reshape(1, 2))


# speedup vs baseline: 1.0273x; 1.0273x over previous
import functools

import jax
import jax.numpy as jnp
from jax import lax
from jax.experimental import pallas as pl
from jax.experimental.pallas import tpu as pltpu
from jax.experimental.pallas import tpu_sc as plsc

B = 4
N = 8192
E = 131072
D = 128
DC = 64
NC, NS = 2, 16
BLK = 2048

C_AGG = 128
EP = E // NS
NCH_E = EP // C_AGG
RT = N // NS
NCH_EMB = 16

_MESH = plsc.VectorSubcoreMesh(
    core_axis_name="c", subcore_axis_name="s", num_cores=NC, num_subcores=NS)
_SC_PARAMS = pltpu.CompilerParams(use_tc_tiling_on_sc=False)

f32 = jnp.float32
i32 = jnp.int32


def _zero_fill(zbuf, rows, width):
    zero16 = jnp.zeros((16,), f32)

    def zrow(r, carry):
        for cc in range(width // 16):
            zbuf[r, pl.ds(cc * 16, 16)] = zero16
        return carry

    lax.fori_loop(0, rows, zrow, 0)


def _zero_accum(agg_s, zbuf, s, zrows):
    for i in range(RT // zrows):
        pltpu.sync_copy(zbuf, agg_s.at[pl.ds(s * RT + i * zrows, zrows)])


def _agg_pass(h_hbm, agg_s, soff, didx, gbufa, gbufb, sga, sgb, ssa, ssb,
              deg_tap=None):
    del ssa, ssb
    pltpu.async_copy(h_hbm.at[soff.at[0]], gbufa, sga)

    def epair(p, carry):
        i0 = 2 * p
        pltpu.async_copy(h_hbm.at[soff.at[i0 + 1]], gbufb, sgb)
        pltpu.make_async_copy(h_hbm.at[soff.at[i0]], gbufa, sga).wait()
        pltpu.sync_copy(gbufa, agg_s.at[didx.at[i0]], add=True)
        if deg_tap is not None:
            deg_tap(i0)

        @pl.when(i0 + 2 < NCH_E)
        def _():
            pltpu.async_copy(h_hbm.at[soff.at[i0 + 2]], gbufa, sga)

        pltpu.make_async_copy(h_hbm.at[soff.at[i0 + 1]], gbufb, sgb).wait()
        pltpu.sync_copy(gbufb, agg_s.at[didx.at[i0 + 1]], add=True)
        if deg_tap is not None:
            deg_tap(i0 + 1)
        return carry

    lax.fori_loop(0, NCH_E // 2, epair, 0)


def _addoff(soff, delta):
    def body(t, carry):
        i = t // (C_AGG // 16)
        j = t - i * (C_AGG // 16)
        sl = pl.ds(j * 16, 16)
        soff[i, sl] = soff[i, sl] + delta
        return carry

    lax.fori_loop(0, NCH_E * (C_AGG // 16), body, 0)


@functools.partial(
    pl.kernel,
    out_type=(jax.ShapeDtypeStruct((B * N, DC), f32),
              jax.ShapeDtypeStruct((N, 16), f32),
              jax.ShapeDtypeStruct((B * N, DC), f32)),
    mesh=_MESH,
    scratch_types=[
        pltpu.VMEM((64, C_AGG), i32),
        pltpu.VMEM((64, C_AGG), i32),
        pltpu.VMEM((C_AGG, DC), f32),
        pltpu.VMEM((C_AGG, DC), f32),
        pltpu.VMEM((C_AGG, 16), f32),
        pltpu.VMEM((C_AGG, 16), f32),
        pltpu.VMEM((32, DC), f32),
        pltpu.VMEM((16, DC), f32),
        pltpu.VMEM((C_AGG, 16), f32),
        pltpu.VMEM((64, 16), f32),
        pltpu.VMEM_SHARED((N, DC), f32),
        pltpu.VMEM_SHARED((N, 16), f32),
        pltpu.SemaphoreType.DMA,
        pltpu.SemaphoreType.DMA,
        pltpu.SemaphoreType.DMA,
        pltpu.SemaphoreType.DMA,
    ],
    compiler_params=_SC_PARAMS,
)
def _embed_agg0(emb_hbm, xl_hbm, ei_hbm, xcat_hbm, deg_hbm, agg_hbm,
                soff, didx, gbufa, gbufb, ebufa, ebufb, ebuf2, zbuf,
                ones_v, zdeg, agg_s, deg_s, sema, semb, semc, semd):
    c = lax.axis_index("c")
    s = lax.axis_index("s")

    _zero_fill(zbuf, 16, DC)
    _zero_accum(agg_s, zbuf, s, 16)
    pltpu.sync_copy(ei_hbm.at[1, s], didx)

    @pl.when(c == 0)
    def _():
        ones16 = jnp.ones((16,), f32)
        zero16 = jnp.zeros((16,), f32)

        def fill(r, carry):
            ones_v[r, pl.ds(0, 16)] = ones16
            return carry

        lax.fori_loop(0, C_AGG, fill, 0)

        def zfill(r, carry):
            zdeg[r, pl.ds(0, 16)] = zero16
            return carry

        lax.fori_loop(0, 64, zfill, 0)
        for i in range(RT // 64):
            pltpu.sync_copy(zdeg, deg_s.at[pl.ds(s * RT + i * 64, 64)])

    plsc.subcore_barrier()

    for local_b in range(2):
        b = 2 * c + local_b
        boff = b * N

        pltpu.sync_copy(xl_hbm.at[b, s], soff.at[pl.ds(0, 16)])

        def estep(i, ebuf, sem):
            pltpu.make_async_copy(
                emb_hbm.at[soff.at[i]], ebuf, sem).wait()

            def rl(t, carry):
                r = t // 4
                cc = t - r * 4
                ebuf2[r, pl.ds(cc * 16, 16)] = ebuf[t, pl.ds(0, 16)]
                return carry

            lax.fori_loop(0, C_AGG, rl, 0)
            pltpu.sync_copy(
                ebuf2, xcat_hbm.at[pl.ds(boff + s * RT + i * 32, 32)])

        pltpu.async_copy(emb_hbm.at[soff.at[0]], ebufa, sema)

        def embpair(p, carry):
            i0 = 2 * p
            pltpu.async_copy(emb_hbm.at[soff.at[i0 + 1]], ebufb, semb)
            estep(i0, ebufa, sema)

            @pl.when(i0 + 2 < NCH_EMB)
            def _():
                pltpu.async_copy(emb_hbm.at[soff.at[i0 + 2]], ebufa, sema)

            estep(i0 + 1, ebufb, semb)
            return carry

        lax.fori_loop(0, NCH_EMB // 2, embpair, 0)
        plsc.subcore_barrier()

        pltpu.sync_copy(ei_hbm.at[0, s], soff)
        _addoff(soff, boff)

        if local_b == 0:
            def deg_tap(i):
                @pl.when(c == 0)
                def _():
                    pltpu.sync_copy(ones_v, deg_s.at[didx.at[i]], add=True)
        else:
            deg_tap = None
        _agg_pass(xcat_hbm, agg_s, soff, didx, gbufa, gbufb, sema, semb,
                  semc, semd, deg_tap)
        plsc.subcore_barrier()
        pltpu.sync_copy(agg_s.at[pl.ds(s * RT, RT)],
                        agg_hbm.at[pl.ds(boff + s * RT, RT)])
        if local_b == 0:
            @pl.when(c == 0)
            def _():
                pltpu.sync_copy(deg_s.at[pl.ds(s * RT, RT)],
                                deg_hbm.at[pl.ds(s * RT, RT)])
            _zero_accum(agg_s, zbuf, s, 16)
            plsc.subcore_barrier()


@functools.partial(
    pl.kernel,
    out_type=jax.ShapeDtypeStruct((B * N, D), f32),
    mesh=_MESH,
    scratch_types=[
        pltpu.VMEM((64, C_AGG), i32),
        pltpu.VMEM((64, C_AGG), i32),
        pltpu.VMEM((C_AGG, D), f32),
        pltpu.VMEM((C_AGG, D), f32),
        pltpu.VMEM((16, D), f32),
        pltpu.VMEM_SHARED((N, D), f32),
        pltpu.SemaphoreType.DMA,
        pltpu.SemaphoreType.DMA,
        pltpu.SemaphoreType.DMA,
        pltpu.SemaphoreType.DMA,
    ],
    compiler_params=_SC_PARAMS,
)
def _agg(h_hbm, ei_hbm, agg_hbm, soff, didx, gbufa, gbufb, zbuf,
         agg_s, sema, semb, semc, semd):
    c = lax.axis_index("c")
    s = lax.axis_index("s")

    _zero_fill(zbuf, 16, D)
    _zero_accum(agg_s, zbuf, s, 16)
    pltpu.sync_copy(ei_hbm.at[0, s], soff)
    pltpu.sync_copy(ei_hbm.at[1, s], didx)
    plsc.subcore_barrier()

    for local_b in range(2):
        b = 2 * c + local_b
        boff = b * N
        _addoff(soff, boff if local_b == 0 else N)
        _agg_pass(h_hbm, agg_s, soff, didx, gbufa, gbufb, sema, semb,
                  semc, semd)
        plsc.subcore_barrier()
        pltpu.sync_copy(agg_s.at[pl.ds(s * RT, RT)],
                        agg_hbm.at[pl.ds(boff + s * RT, RT)])
        if local_b == 0:
            _zero_accum(agg_s, zbuf, s, 16)
            plsc.subcore_barrier()


def _layer0_body(xc_ref, agg_ref, deg_ref, w64_ref, wrole_ref, role1_ref,
                 linb_ref, wl_ref, bl_ref, wr_ref, out_ref, inv_ref):
    degcol = deg_ref[...][:, 0:1]
    inv = 1.0 / jnp.maximum(degcol, 1.0)
    gate = jnp.where(degcol > 0.0, 1.0, 0.0)
    beff = linb_ref[...] + jnp.dot(
        role1_ref[...], wrole_ref[...], preferred_element_type=f32)
    mean_c = agg_ref[...] * inv
    x0 = jnp.dot(xc_ref[...], w64_ref[...], preferred_element_type=f32) + beff
    m0 = (jnp.dot(mean_c, w64_ref[...], preferred_element_type=f32)
          + gate * beff)
    h = (jnp.dot(m0, wl_ref[...], preferred_element_type=f32) + bl_ref[...]
         + jnp.dot(x0, wr_ref[...], preferred_element_type=f32))
    out_ref[...] = jnp.maximum(h, 0.0)
    inv_ref[...] = jnp.broadcast_to(inv, (BLK, D))


def _layer0(xc, agg, deg2, w64T, wroleT, role1, linb2, wlT, bl2, wrT):
    nb = (B * N) // BLK
    return pl.pallas_call(
        _layer0_body,
        grid=(nb,),
        in_specs=[
            pl.BlockSpec((BLK, DC), lambda j: (j, 0)),
            pl.BlockSpec((BLK, DC), lambda j: (j, 0)),
            pl.BlockSpec((BLK, 16), lambda j: (lax.rem(j, N // BLK), 0)),
            pl.BlockSpec((DC, D), lambda j: (0, 0)),
            pl.BlockSpec((16, D), lambda j: (0, 0)),
            pl.BlockSpec((1, 16), lambda j: (0, 0)),
            pl.BlockSpec((1, D), lambda j: (0, 0)),
            pl.BlockSpec((D, D), lambda j: (0, 0)),
            pl.BlockSpec((1, D), lambda j: (0, 0)),
            pl.BlockSpec((D, D), lambda j: (0, 0)),
        ],
        out_specs=[
            pl.BlockSpec((BLK, D), lambda j: (j, 0)),
            pl.BlockSpec((BLK, D), lambda j: (lax.rem(j, N // BLK), 0)),
        ],
        out_shape=[jax.ShapeDtypeStruct((B * N, D), f32),
                   jax.ShapeDtypeStruct((N, D), f32)],
    )(xc, agg, deg2, w64T, wroleT, role1, linb2, wlT, bl2, wrT)


def _layer1_head_body(x_ref, agg_ref, inv_ref, wl_ref, bl_ref, wr_ref,
                      w1_ref, b1_ref, w2_ref, b2_ref, w3_ref, b3_ref,
                      out_ref):
    mean = agg_ref[...] * inv_ref[...]
    h = (jnp.dot(mean, wl_ref[...], preferred_element_type=f32) + bl_ref[...]
         + jnp.dot(x_ref[...], wr_ref[...], preferred_element_type=f32))
    h = jnp.maximum(h, 0.0)
    h = jnp.maximum(
        jnp.dot(h, w1_ref[...], preferred_element_type=f32) + b1_ref[...], 0.0)
    h = jnp.maximum(
        jnp.dot(h, w2_ref[...], preferred_element_type=f32) + b2_ref[...], 0.0)
    out = (jnp.dot(h, w3_ref[...], preferred_element_type=f32)
           + b3_ref[...])
    out_ref[...] = out.T.reshape(1, 2, BLK)


def _layer1_head(x, agg, invb, wlT, bl2, wrT, w1T, b12, w2T, b22, w3T, b32):
    nbn = N // BLK
    return pl.pallas_call(
        _layer1_head_body,
        grid=(B, nbn),
        in_specs=[
            pl.BlockSpec((BLK, D), lambda b, j: (b * nbn + j, 0)),
            pl.BlockSpec((BLK, D), lambda b, j: (b * nbn + j, 0)),
            pl.BlockSpec((BLK, D), lambda b, j: (j, 0)),
            pl.BlockSpec((D, D), lambda b, j: (0, 0)),
            pl.BlockSpec((1, D), lambda b, j: (0, 0)),
            pl.BlockSpec((D, D), lambda b, j: (0, 0)),
            pl.BlockSpec((D, D), lambda b, j: (0, 0)),
            pl.BlockSpec((1, D), lambda b, j: (0, 0)),
            pl.BlockSpec((D, D), lambda b, j: (0, 0)),
            pl.BlockSpec((1, D), lambda b, j: (0, 0)),
            pl.BlockSpec((D, 2), lambda b, j: (0, 0)),
            pl.BlockSpec((1, 2), lambda b, j: (0, 0)),
        ],
        out_specs=pl.BlockSpec((1, 2, BLK), lambda b, j: (b, 0, j)),
        out_shape=jax.ShapeDtypeStruct((B, 2, N), f32),
    )(x, agg, invb, wlT, bl2, wrT, w1T, b12, w2T, b22, w3T, b32)


def kernel(x_layout, x_role, edge_index, role_emb, layout_emb, lin_W, lin_b,
           c0_Wl, c0_bl, c0_Wr, c1_Wl, c1_bl, c1_Wr,
           d1_W, d1_b, d2_W, d2_b, d3_W, d3_b):
    del x_role
    xl3 = x_layout.reshape(B, NS, NCH_EMB, C_AGG)
    ei4 = edge_index.reshape(2, NS, NCH_E, C_AGG)

    xcat, deg2, agg0 = _embed_agg0(layout_emb, xl3, ei4)
    h1, invb = _layer0(xcat, agg0, deg2, lin_W[:, :DC].T, lin_W[:, DC:80].T,
                       role_emb[1:2, :], lin_b.reshape(1, D),
                       c0_Wl.T, c0_bl.reshape(1, D), c0_Wr.T)
    agg1 = _agg(h1, ei4)
    out = _layer1_head(h1, agg1, invb, c1_Wl.T, c1_bl.reshape(1, D),
                       c1_Wr.T, d1_W.T, d1_b.reshape(1, D), d2_W.T,
                       d2_b.reshape(1, D), d3_W.T, d3_b.reshape(1, 2))
    return jnp.swapaxes(out, 1, 2)
